# Initial kernel scaffold; baseline (speedup 1.0000x reference)
import functools

import jax
import jax.numpy as jnp
from jax import lax
from jax.experimental import pallas as pl
from jax.experimental.pallas import tpu as pltpu
from jax.experimental.pallas import tpu_sc as plsc

N = 10000
E = 320000
D = 128

NC = 2
NS = 16
NW = NC * NS
G = 128
GPW = 79
EP = NW * GPW * G
ROWS_PER_TILE = N // NS

_mesh = plsc.VectorSubcoreMesh(core_axis_name="c", subcore_axis_name="s")


@functools.partial(
    pl.kernel,
    out_type=jax.ShapeDtypeStruct((NC, N, D), jnp.float32),
    mesh=_mesh,
    scratch_types=[
        pltpu.VMEM((GPW, G), jnp.int32),
        pltpu.VMEM((GPW, G), jnp.int32),
        pltpu.VMEM((GPW, G), jnp.float32),
        pltpu.VMEM((G, D), jnp.float32),
        pltpu.VMEM_SHARED((N, D), jnp.float32),
        pltpu.SemaphoreType.DMA,
    ],
)
def _spmm_sc(pre_hbm, cols_hbm, rows_hbm, vals_hbm, out_hbm,
             cols_v, rows_v, vals_v, gbuf, acc, sem):
    c = lax.axis_index("c")
    s = lax.axis_index("s")
    wid = s * NC + c

    pltpu.sync_copy(cols_hbm.at[wid], cols_v)
    pltpu.sync_copy(rows_hbm.at[wid], rows_v)
    pltpu.sync_copy(vals_hbm.at[wid], vals_v)

    zero = jnp.zeros((16,), jnp.float32)

    def _zrow(r, carry):
        for j in range(D // 16):
            gbuf[r, pl.ds(16 * j, 16)] = zero
        return carry

    lax.fori_loop(0, G, _zrow, 0)
    for k in range(5):
        pltpu.sync_copy(
            gbuf.at[pl.ds(0, 125)],
            acc.at[pl.ds(s * ROWS_PER_TILE + k * 125, 125)],
        )
    plsc.subcore_barrier()

    def _group(g, carry):
        pltpu.async_copy(pre_hbm.at[cols_v.at[g]], gbuf, sem).wait()

        def _edge(e, carry2):
            v = vals_v[g, e]
            for j in range(D // 16):
                sl = pl.ds(16 * j, 16)
                gbuf[e, sl] = gbuf[e, sl] * v
            return carry2

        lax.fori_loop(0, G, _edge, 0)
        pltpu.sync_copy(gbuf, acc.at[rows_v.at[g]], add=True)
        return carry

    lax.fori_loop(0, GPW, _group, 0)
    plsc.subcore_barrier()

    pltpu.sync_copy(
        acc.at[pl.ds(s * ROWS_PER_TILE, ROWS_PER_TILE)],
        out_hbm.at[c, pl.ds(s * ROWS_PER_TILE, ROWS_PER_TILE)],
    )


def _mm_body(x_ref, w_ref, o_ref):
    o_ref[...] = jnp.dot(x_ref[...], w_ref[...],
                         preferred_element_type=jnp.float32)


_matmul = pl.pallas_call(
    _mm_body,
    grid=(8,),
    in_specs=[
        pl.BlockSpec((N // 8, D), lambda i: (i, 0)),
        pl.BlockSpec((D, D), lambda i: (0, 0)),
    ],
    out_specs=pl.BlockSpec((N // 8, D), lambda i: (i, 0)),
    out_shape=jax.ShapeDtypeStruct((N, D), jnp.float32),
)


def _add_body(p_ref, o_ref):
    o_ref[...] = p_ref[0] + p_ref[1]


_add_partials = pl.pallas_call(
    _add_body,
    grid=(8,),
    in_specs=[pl.BlockSpec((NC, N // 8, D), lambda i: (0, i, 0))],
    out_specs=pl.BlockSpec((N // 8, D), lambda i: (i, 0)),
    out_shape=jax.ShapeDtypeStruct((N, D), jnp.float32),
)


def kernel(x, adj_indices, adj_values, W):
    pre_sup = _matmul(x, W)
    pad = EP - E
    cols = jnp.pad(adj_indices[1], (0, pad)).reshape(NW, GPW, G)
    rows = jnp.pad(adj_indices[0], (0, pad)).reshape(NW, GPW, G)
    vals = jnp.pad(adj_values, (0, pad)).reshape(NW, GPW, G)
    partials = _spmm_sc(pre_sup, cols, rows, vals)
    return _add_partials(partials)


# speedup vs baseline: 4.6547x; 4.6547x over previous
import functools

import jax
import jax.numpy as jnp
from jax import lax
from jax.experimental import pallas as pl
from jax.experimental.pallas import tpu as pltpu
from jax.experimental.pallas import tpu_sc as plsc

N = 10000
E = 320000
D = 128

NC = 2
NS = 16
NW = NC * NS
G = 128
GPW = 79
EP = NW * GPW * G
ROW_STRIDE = 624
ROW_SPAN = 640

_mesh = plsc.VectorSubcoreMesh(core_axis_name="c", subcore_axis_name="s")


@functools.partial(
    pl.kernel,
    out_type=jax.ShapeDtypeStruct((NC, N, D), jnp.float32),
    mesh=_mesh,
    scratch_types=[
        pltpu.VMEM((GPW, G), jnp.int32),
        pltpu.VMEM((GPW, G), jnp.int32),
        pltpu.VMEM((GPW, G), jnp.float32),
        pltpu.VMEM((G, D), jnp.float32),
        pltpu.VMEM_SHARED((N, D), jnp.float32),
        pltpu.SemaphoreType.DMA,
    ],
)
def _spmm_sc(pre_hbm, cols_hbm, rows_hbm, vals_hbm, out_hbm,
             cols_v, rows_v, vals_v, gbuf, acc, sem):
    c = lax.axis_index("c")
    s = lax.axis_index("s")
    wid = s * NC + c

    pltpu.sync_copy(cols_hbm.at[wid], cols_v)
    pltpu.sync_copy(rows_hbm.at[wid], rows_v)
    pltpu.sync_copy(vals_hbm.at[wid], vals_v)

    zero = jnp.zeros((16,), jnp.float32)

    def _zrow(r, carry):
        for j in range(D // 16):
            gbuf[r, pl.ds(16 * j, 16)] = zero
        return carry

    lax.fori_loop(0, G, _zrow, 0)
    for k in range(ROW_SPAN // G):
        pltpu.sync_copy(
            gbuf,
            acc.at[pl.ds(s * ROW_STRIDE + k * G, G)],
        )
    plsc.subcore_barrier()

    def _group(g, carry):
        pltpu.async_copy(pre_hbm.at[cols_v.at[g]], gbuf, sem).wait()

        def _eblock(eb, carry2):
            vvec = vals_v[g, pl.ds(eb * 16, 16)]
            for l in range(16):
                v = vvec[l]
                e = eb * 16 + l
                for j in range(D // 16):
                    sl = pl.ds(16 * j, 16)
                    gbuf[e, sl] = gbuf[e, sl] * v
            return carry2

        lax.fori_loop(0, G // 16, _eblock, 0)
        pltpu.sync_copy(gbuf, acc.at[rows_v.at[g]], add=True)
        return carry

    lax.fori_loop(0, GPW, _group, 0)
    plsc.subcore_barrier()

    pltpu.sync_copy(
        acc.at[pl.ds(s * ROW_STRIDE, ROW_SPAN)],
        out_hbm.at[c, pl.ds(s * ROW_STRIDE, ROW_SPAN)],
    )


def _mm_body(x_ref, w_ref, o_ref):
    o_ref[...] = jnp.dot(x_ref[...], w_ref[...],
                         preferred_element_type=jnp.float32)


_matmul = pl.pallas_call(
    _mm_body,
    grid=(10,),
    in_specs=[
        pl.BlockSpec((N // 10, D), lambda i: (i, 0)),
        pl.BlockSpec((D, D), lambda i: (0, 0)),
    ],
    out_specs=pl.BlockSpec((N // 10, D), lambda i: (i, 0)),
    out_shape=jax.ShapeDtypeStruct((N, D), jnp.float32),
)


def _add_body(p_ref, o_ref):
    o_ref[...] = p_ref[0] + p_ref[1]


_add_partials = pl.pallas_call(
    _add_body,
    grid=(10,),
    in_specs=[pl.BlockSpec((NC, N // 10, D), lambda i: (0, i, 0))],
    out_specs=pl.BlockSpec((N // 10, D), lambda i: (i, 0)),
    out_shape=jax.ShapeDtypeStruct((N, D), jnp.float32),
)


def kernel(x, adj_indices, adj_values, W):
    pre_sup = _matmul(x, W)
    pad = EP - E
    cols = jnp.pad(adj_indices[1], (0, pad)).reshape(NW, GPW, G)
    rows = jnp.pad(adj_indices[0], (0, pad)).reshape(NW, GPW, G)
    vals = jnp.pad(adj_values, (0, pad)).reshape(NW, GPW, G)
    partials = _spmm_sc(pre_sup, cols, rows, vals)
    return _add_partials(partials)


# speedup vs baseline: 6.5999x; 1.4179x over previous
import functools

import jax
import jax.numpy as jnp
from jax import lax
from jax.experimental import pallas as pl
from jax.experimental.pallas import tpu as pltpu
from jax.experimental.pallas import tpu_sc as plsc

N = 10000
E = 320000
D = 128

NC = 2
NS = 16
NW = NC * NS
G = 112
CH = 6
NCHUNK = 15
GPW = CH * NCHUNK
EP = NW * GPW * G

ROW_STRIDE = 624
ROW_SPAN = 640

_mesh = plsc.VectorSubcoreMesh(core_axis_name="c", subcore_axis_name="s")


@functools.partial(
    pl.kernel,
    out_type=jax.ShapeDtypeStruct((NC, N, D), jnp.float32),
    mesh=_mesh,
    scratch_types=[
        pltpu.VMEM((2, CH, G), jnp.int32),
        pltpu.VMEM((2, CH, G), jnp.int32),
        pltpu.VMEM((2, CH, G), jnp.float32),
        pltpu.VMEM((G, D), jnp.float32),
        pltpu.VMEM((G, D), jnp.float32),
        pltpu.VMEM((G, D), jnp.float32),
        pltpu.VMEM_SHARED((N, D), jnp.float32),
        pltpu.SemaphoreType.DMA,
        pltpu.SemaphoreType.DMA,
        pltpu.SemaphoreType.DMA,
        pltpu.SemaphoreType.DMA,
        pltpu.SemaphoreType.DMA,
        pltpu.SemaphoreType.DMA,
        pltpu.SemaphoreType.DMA,
    ],
)
def _spmm_sc(pre_hbm, cols_hbm, rows_hbm, vals_hbm, out_hbm,
             colsb, rowsb, valsb, gb0, gb1, gb2, acc,
             sg0, sg1, sg2, ss0, ss1, ss2, si):
    c = lax.axis_index("c")
    s = lax.axis_index("s")
    wid = s * NC + c
    bufs = (gb0, gb1, gb2)
    semg = (sg0, sg1, sg2)
    sems = (ss0, ss1, ss2)

    zero = jnp.zeros((16,), jnp.float32)

    def _zrow(r, carry):
        for j in range(D // 16):
            gb0[r, pl.ds(16 * j, 16)] = zero
        return carry

    lax.fori_loop(0, 80, _zrow, 0)
    for k in range(ROW_SPAN // 80):
        pltpu.sync_copy(
            gb0.at[pl.ds(0, 80)],
            acc.at[pl.ds(s * ROW_STRIDE + k * 80, 80)],
        )
    plsc.subcore_barrier()

    def _fetch_idx(chunk, slot):
        pltpu.async_copy(cols_hbm.at[wid, chunk], colsb.at[slot], si)
        pltpu.async_copy(rows_hbm.at[wid, chunk], rowsb.at[slot], si)
        pltpu.async_copy(vals_hbm.at[wid, chunk], valsb.at[slot], si)

    def _wait_idx():
        pltpu.make_async_copy(cols_hbm.at[wid, 0], colsb.at[0], si).wait()
        pltpu.make_async_copy(rows_hbm.at[wid, 0], rowsb.at[0], si).wait()
        pltpu.make_async_copy(vals_hbm.at[wid, 0], valsb.at[0], si).wait()

    def _start_gather(buf, sem, slot, pos):
        pltpu.async_copy(pre_hbm.at[colsb.at[slot, pos]], buf, sem)

    def _wait_gather(buf, sem):
        pltpu.make_async_copy(pre_hbm.at[pl.ds(0, G)], buf, sem).wait()

    def _start_scatter(buf, sem, slot, pos):
        pltpu.async_copy(buf, acc.at[rowsb.at[slot, pos]], sem, add=True)

    def _wait_scatter(buf, sem):
        pltpu.make_async_copy(buf, acc.at[pl.ds(0, G)], sem).wait()

    def _scale(buf, slot, pos):
        def _eblock(eb, carry):
            vvec = valsb[slot, pos, pl.ds(eb * 16, 16)]
            for l in range(16):
                v = vvec[l]
                e = eb * 16 + l
                for j in range(D // 16):
                    sl = pl.ds(16 * j, 16)
                    buf[e, sl] = buf[e, sl] * v
            return carry

        lax.fori_loop(0, G // 16, _eblock, 0)

    def _body(slot, next_slot, k, first_chunk=False, last_chunk=False):
        bk = k % 3
        bj = (k + 1) % 3
        _wait_gather(bufs[bk], semg[bk])
        if not (first_chunk and k < 2):
            _wait_scatter(bufs[bj], sems[bj])
        if k == CH - 1:
            if not last_chunk:
                _wait_idx()
                _start_gather(bufs[bj], semg[bj], next_slot, 0)
        else:
            _start_gather(bufs[bj], semg[bj], slot, k + 1)
        _scale(bufs[bk], slot, k)
        _start_scatter(bufs[bk], sems[bk], slot, k)

    _fetch_idx(0, 0)
    _wait_idx()
    _start_gather(gb0, sg0, 0, 0)
    _fetch_idx(1, 1)
    for k in range(CH):
        _body(0, 1, k, first_chunk=True)

    def _super(ci, carry):
        slot = lax.rem(ci, 2)
        next_slot = lax.rem(ci + 1, 2)
        _fetch_idx(ci + 1, next_slot)
        for k in range(CH):
            _body(slot, next_slot, k)
        return carry

    lax.fori_loop(1, NCHUNK - 1, _super, 0)

    lslot = (NCHUNK - 1) % 2
    for k in range(CH):
        _body(lslot, 1 - lslot, k, last_chunk=True)
    _wait_scatter(bufs[(CH - 2) % 3], sems[(CH - 2) % 3])
    _wait_scatter(bufs[(CH - 1) % 3], sems[(CH - 1) % 3])
    plsc.subcore_barrier()

    pltpu.sync_copy(
        acc.at[pl.ds(s * ROW_STRIDE, ROW_SPAN)],
        out_hbm.at[c, pl.ds(s * ROW_STRIDE, ROW_SPAN)],
    )


def _mm_body(x_ref, w_ref, o_ref):
    o_ref[...] = jnp.dot(x_ref[...], w_ref[...],
                         preferred_element_type=jnp.float32)


_matmul = pl.pallas_call(
    _mm_body,
    grid=(10,),
    in_specs=[
        pl.BlockSpec((N // 10, D), lambda i: (i, 0)),
        pl.BlockSpec((D, D), lambda i: (0, 0)),
    ],
    out_specs=pl.BlockSpec((N // 10, D), lambda i: (i, 0)),
    out_shape=jax.ShapeDtypeStruct((N, D), jnp.float32),
)


def _add_body(p_ref, o_ref):
    o_ref[...] = p_ref[0] + p_ref[1]


_add_partials = pl.pallas_call(
    _add_body,
    grid=(10,),
    in_specs=[pl.BlockSpec((NC, N // 10, D), lambda i: (0, i, 0))],
    out_specs=pl.BlockSpec((N // 10, D), lambda i: (i, 0)),
    out_shape=jax.ShapeDtypeStruct((N, D), jnp.float32),
)


def kernel(x, adj_indices, adj_values, W):
    pre_sup = _matmul(x, W)
    pad = EP - E
    cols = jnp.pad(adj_indices[1], (0, pad)).reshape(NW, NCHUNK, CH, G)
    rows = jnp.pad(adj_indices[0], (0, pad)).reshape(NW, NCHUNK, CH, G)
    vals = jnp.pad(adj_values, (0, pad)).reshape(NW, NCHUNK, CH, G)
    partials = _spmm_sc(pre_sup, cols, rows, vals)
    return _add_partials(partials)
